# baseline (speedup 1.0000x reference)
import functools

import jax
import jax.numpy as jnp
from jax import lax
from jax.experimental import pallas as pl
from jax.experimental.pallas import tpu as pltpu
from jax.experimental.pallas import tpu_sc as plsc

N = 10000
D = 128
E = 320000
NC = 2
NS = 16
NW = NC * NS
EPW = E // NW
WIN = 128
SEGW = 16
NSEG = 5
EPT = NSEG * SEGW * WIN
PADT = EPT - EPW
NSINK = 48
NA = N + NSINK
CH = 80
NCH = N // CH
CPS = -(-NCH // NS)

_mesh = plsc.VectorSubcoreMesh(core_axis_name="c", subcore_axis_name="s")


def _fill(ref, rows, cols, val):
    v = jnp.full((16,), val, jnp.float32)

    @pl.loop(0, rows)
    def _(i):
        @pl.loop(0, cols, step=16)
        def _(j):
            ref[i, pl.ds(j, 16)] = v


def _chunk_loop(sid, body):
    @pl.loop(0, CPS)
    def _(k):
        c = k * NS + sid

        @pl.when(c < NCH)
        def _():
            body(c * CH)


@functools.partial(
    pl.kernel,
    out_type=jax.ShapeDtypeStruct((NC, N, D), jnp.float32),
    mesh=_mesh,
    scratch_types=[
        pltpu.VMEM((2, SEGW, WIN), jnp.int32),
        pltpu.VMEM((WIN, D), jnp.float32),
        pltpu.VMEM((CH, D), jnp.float32),
        pltpu.VMEM_SHARED((NA, D), jnp.float32),
    ],
)
def _hist_kernel(edge_hbm, out_hbm, idx_v, ones_v, zrows_v, acc_sh):
    cid = lax.axis_index("c")
    sid = lax.axis_index("s")
    wid = cid * NS + sid
    _fill(zrows_v, CH, D, 0.0)
    _fill(ones_v, WIN, D, 1.0)
    _chunk_loop(sid, lambda r: pltpu.sync_copy(zrows_v, acc_sh.at[pl.ds(r, CH)]))
    plsc.subcore_barrier()

    @pl.loop(0, NSEG)
    def _(s):
        pltpu.sync_copy(edge_hbm.at[wid].at[s], idx_v)

        @pl.loop(0, SEGW)
        def _(j):
            pltpu.sync_copy(ones_v, acc_sh.at[idx_v.at[1].at[j]], add=True)

    plsc.subcore_barrier()
    _chunk_loop(sid, lambda r: pltpu.sync_copy(
        acc_sh.at[pl.ds(r, CH)], out_hbm.at[cid].at[pl.ds(r, CH)]))


@functools.partial(
    pl.kernel,
    out_type=jax.ShapeDtypeStruct((NC, N, D), jnp.float32),
    mesh=_mesh,
    scratch_types=[
        pltpu.VMEM((2, SEGW, WIN), jnp.int32),
        pltpu.VMEM((2, WIN, D), jnp.float32),
        pltpu.VMEM_SHARED((NA, D), jnp.float32),
        pltpu.SemaphoreType.DMA,
        pltpu.SemaphoreType.DMA,
    ],
)
def _agg_kernel(y_hbm, edge_hbm, out_hbm, idx_v, bufs_v, acc_sh, sema, semb):
    cid = lax.axis_index("c")
    sid = lax.axis_index("s")
    wid = cid * NS + sid
    bufa_v = bufs_v.at[0]
    bufb_v = bufs_v.at[1]
    v0 = jnp.zeros((16,), jnp.float32)

    @pl.loop(0, CH)
    def _(i):
        @pl.loop(0, D, step=16)
        def _(j):
            bufs_v[0, i, pl.ds(j, 16)] = v0

    zsrc = bufa_v.at[pl.ds(0, CH)]
    _chunk_loop(sid, lambda r: pltpu.sync_copy(zsrc, acc_sh.at[pl.ds(r, CH)]))
    plsc.subcore_barrier()

    @pl.loop(0, NSEG)
    def _(s):
        pltpu.sync_copy(edge_hbm.at[wid].at[s], idx_v)

        @pl.loop(0, SEGW)
        def _(j):
            pltpu.sync_copy(y_hbm.at[idx_v.at[0].at[j]], bufa_v)
            pltpu.sync_copy(bufa_v, acc_sh.at[idx_v.at[1].at[j]], add=True)

    plsc.subcore_barrier()
    _chunk_loop(sid, lambda r: pltpu.sync_copy(
        acc_sh.at[pl.ds(r, CH)], out_hbm.at[cid].at[pl.ds(r, CH)]))


def _dot(a, b):
    return lax.dot_general(a, b, (((1,), (0,)), ((), ())),
                           precision=lax.Precision.HIGHEST,
                           preferred_element_type=jnp.float32)


def _dis_from_hist(hist_ref):
    deg = hist_ref[0, :, 0:1] + hist_ref[1, :, 0:1] + 1.0
    return lax.rsqrt(deg)


def _mm_body(x_ref, w_ref, o_ref):
    o_ref[...] = _dot(x_ref[...], w_ref[...])


def _scale_body(hist_ref, xw_ref, o_ref):
    o_ref[...] = xw_ref[...] * _dis_from_hist(hist_ref)


def _mid_body(hist_ref, s_ref, y_ref, w_ref, b_ref, o_ref):
    dis = _dis_from_hist(hist_ref)
    h = jnp.maximum(dis * (s_ref[0] + s_ref[1] + y_ref[...]) + b_ref[...], 0.0)
    o_ref[...] = dis * _dot(h, w_ref[...])


def _final_body(hist_ref, s_ref, y_ref, b_ref, o_ref):
    dis = _dis_from_hist(hist_ref)
    o_ref[...] = dis * (s_ref[0] + s_ref[1] + y_ref[...]) + b_ref[...]


_nd_f32 = jax.ShapeDtypeStruct((N, D), jnp.float32)


def _pack_edges(edge_index):
    rc = edge_index.astype(jnp.int32).reshape(2, NW, EPW)
    pad_r = jnp.zeros((1, NW, PADT), jnp.int32)
    sink = N + jnp.arange(PADT, dtype=jnp.int32) % NSINK
    pad_c = jnp.broadcast_to(sink, (1, NW, PADT))
    rcp = jnp.concatenate([rc, jnp.concatenate([pad_r, pad_c], 0)], 2)
    return rcp.reshape(2, NW, NSEG, SEGW, WIN).transpose(1, 2, 0, 3, 4)


def kernel(x, edge_index, W1, b1, W2, b2):
    edges = _pack_edges(edge_index)
    b1r = b1.reshape(1, D)
    b2r = b2.reshape(1, D)

    hist = _hist_kernel(edges)
    xw1 = pl.pallas_call(_mm_body, out_shape=_nd_f32)(x, W1)
    y1 = pl.pallas_call(_scale_body, out_shape=_nd_f32)(hist, xw1)
    s1 = _agg_kernel(y1, edges)
    y2 = pl.pallas_call(_mid_body, out_shape=_nd_f32)(hist, s1, y1, W2, b1r)
    s2 = _agg_kernel(y2, edges)
    out = pl.pallas_call(_final_body, out_shape=_nd_f32)(hist, s2, y2, b2r)
    return out


# speedup vs baseline: 1.9180x; 1.9180x over previous
import functools

import jax
import jax.numpy as jnp
from jax import lax
from jax.experimental import pallas as pl
from jax.experimental.pallas import tpu as pltpu
from jax.experimental.pallas import tpu_sc as plsc

N = 10000
D = 128
E = 320000
NC = 2
NS = 16
NW = NC * NS
EPW = E // NW
WIN = 80
SEGW = 25
NSEG = 5
CH = 80
NCH = N // CH
CPS = -(-NCH // NS)

_mesh = plsc.VectorSubcoreMesh(core_axis_name="c", subcore_axis_name="s")


def _fill(ref, rows, cols, val):
    v = jnp.full((16,), val, jnp.float32)

    @pl.loop(0, rows)
    def _(i):
        @pl.loop(0, cols, step=16)
        def _(j):
            ref[i, pl.ds(j, 16)] = v


def _chunk_loop(sid, body):
    @pl.loop(0, CPS)
    def _(k):
        c = k * NS + sid

        @pl.when(c < NCH)
        def _():
            body(c * CH)


@functools.partial(
    pl.kernel,
    out_type=jax.ShapeDtypeStruct((NC, N, D), jnp.float32),
    mesh=_mesh,
    scratch_types=[
        pltpu.VMEM((SEGW, WIN), jnp.int32),
        pltpu.VMEM((WIN, D), jnp.float32),
        pltpu.VMEM((CH, D), jnp.float32),
        pltpu.VMEM_SHARED((N, D), jnp.float32),
    ],
)
def _hist_kernel(col_hbm, out_hbm, idx_v, ones_v, zrows_v, acc_sh):
    cid = lax.axis_index("c")
    sid = lax.axis_index("s")
    wid = cid * NS + sid
    _fill(zrows_v, CH, D, 0.0)
    _fill(ones_v, WIN, D, 1.0)
    _chunk_loop(sid, lambda r: pltpu.sync_copy(zrows_v, acc_sh.at[pl.ds(r, CH)]))
    plsc.subcore_barrier()

    @pl.loop(0, NSEG)
    def _(s):
        pltpu.sync_copy(col_hbm.at[wid].at[s], idx_v)

        @pl.loop(0, SEGW)
        def _(j):
            pltpu.sync_copy(ones_v, acc_sh.at[idx_v.at[j]], add=True)

    plsc.subcore_barrier()
    _chunk_loop(sid, lambda r: pltpu.sync_copy(
        acc_sh.at[pl.ds(r, CH)], out_hbm.at[cid].at[pl.ds(r, CH)]))


@functools.partial(
    pl.kernel,
    out_type=jax.ShapeDtypeStruct((NC, N, D), jnp.float32),
    mesh=_mesh,
    scratch_types=[
        pltpu.VMEM((SEGW, WIN), jnp.int32),
        pltpu.VMEM((SEGW, WIN), jnp.int32),
        pltpu.VMEM((WIN, D), jnp.float32),
        pltpu.VMEM((CH, D), jnp.float32),
        pltpu.VMEM_SHARED((N, D), jnp.float32),
    ],
)
def _agg_kernel(y_hbm, row_hbm, col_hbm, out_hbm, idxr_v, idxc_v, buf_v,
                zrows_v, acc_sh):
    cid = lax.axis_index("c")
    sid = lax.axis_index("s")
    wid = cid * NS + sid
    _fill(zrows_v, CH, D, 0.0)
    _chunk_loop(sid, lambda r: pltpu.sync_copy(zrows_v, acc_sh.at[pl.ds(r, CH)]))
    plsc.subcore_barrier()

    @pl.loop(0, NSEG)
    def _(s):
        pltpu.sync_copy(row_hbm.at[wid].at[s], idxr_v)
        pltpu.sync_copy(col_hbm.at[wid].at[s], idxc_v)

        @pl.loop(0, SEGW)
        def _(j):
            pltpu.sync_copy(y_hbm.at[idxr_v.at[j]], buf_v)
            pltpu.sync_copy(buf_v, acc_sh.at[idxc_v.at[j]], add=True)

    plsc.subcore_barrier()
    _chunk_loop(sid, lambda r: pltpu.sync_copy(
        acc_sh.at[pl.ds(r, CH)], out_hbm.at[cid].at[pl.ds(r, CH)]))


def _dot(a, b):
    return lax.dot_general(a, b, (((1,), (0,)), ((), ())),
                           precision=lax.Precision.HIGHEST,
                           preferred_element_type=jnp.float32)


def _dis_from_hist(hist_ref):
    deg = hist_ref[0, :, 0:1] + hist_ref[1, :, 0:1] + 1.0
    return lax.rsqrt(deg)


def _mm_body(x_ref, w_ref, o_ref):
    o_ref[...] = _dot(x_ref[...], w_ref[...])


def _scale_body(hist_ref, xw_ref, o_ref):
    o_ref[...] = xw_ref[...] * _dis_from_hist(hist_ref)


def _mid_body(hist_ref, s_ref, y_ref, w_ref, b_ref, o_ref):
    dis = _dis_from_hist(hist_ref)
    h = jnp.maximum(dis * (s_ref[0] + s_ref[1] + y_ref[...]) + b_ref[...], 0.0)
    o_ref[...] = dis * _dot(h, w_ref[...])


def _final_body(hist_ref, s_ref, y_ref, b_ref, o_ref):
    dis = _dis_from_hist(hist_ref)
    o_ref[...] = dis * (s_ref[0] + s_ref[1] + y_ref[...]) + b_ref[...]


_nd_f32 = jax.ShapeDtypeStruct((N, D), jnp.float32)


def kernel(x, edge_index, W1, b1, W2, b2):
    rc = edge_index.astype(jnp.int32).reshape(2, NW, NSEG, SEGW, WIN)
    row = rc[0]
    col = rc[1]
    b1r = b1.reshape(1, D)
    b2r = b2.reshape(1, D)

    hist = _hist_kernel(col)
    xw1 = pl.pallas_call(_mm_body, out_shape=_nd_f32)(x, W1)
    y1 = pl.pallas_call(_scale_body, out_shape=_nd_f32)(hist, xw1)
    s1 = _agg_kernel(y1, row, col)
    y2 = pl.pallas_call(_mid_body, out_shape=_nd_f32)(hist, s1, y1, W2, b1r)
    s2 = _agg_kernel(y2, row, col)
    out = pl.pallas_call(_final_body, out_shape=_nd_f32)(hist, s2, y2, b2r)
    return out


# speedup vs baseline: 2.7597x; 1.4389x over previous
import functools

import jax
import jax.numpy as jnp
from jax import lax
from jax.experimental import pallas as pl
from jax.experimental.pallas import tpu as pltpu
from jax.experimental.pallas import tpu_sc as plsc

N = 10000
D = 128
E = 320000
NC = 2
NS = 16
NW = NC * NS
EPW = E // NW
WIN = 80
SEGW = 25
NSEG = 5
CH = 80
NCH = N // CH
CPS = -(-NCH // NS)

_mesh = plsc.VectorSubcoreMesh(core_axis_name="c", subcore_axis_name="s")


def _fill(ref, rows, cols, val):
    v = jnp.full((16,), val, jnp.float32)

    @pl.loop(0, rows)
    def _(i):
        @pl.loop(0, cols, step=16)
        def _(j):
            ref[i, pl.ds(j, 16)] = v


def _chunk_loop(sid, body):
    @pl.loop(0, CPS)
    def _(k):
        c = k * NS + sid

        @pl.when(c < NCH)
        def _():
            body(c * CH)


@functools.partial(
    pl.kernel,
    out_type=jax.ShapeDtypeStruct((NC, N, D), jnp.float32),
    mesh=_mesh,
    scratch_types=[
        pltpu.VMEM((SEGW, WIN), jnp.int32),
        pltpu.VMEM((WIN, D), jnp.float32),
        pltpu.VMEM((CH, D), jnp.float32),
        pltpu.VMEM_SHARED((N, D), jnp.float32),
    ],
)
def _hist_kernel(col_hbm, out_hbm, idx_v, ones_v, zrows_v, acc_sh):
    cid = lax.axis_index("c")
    sid = lax.axis_index("s")
    wid = cid * NS + sid
    _fill(zrows_v, CH, D, 0.0)
    _fill(ones_v, WIN, D, 1.0)
    _chunk_loop(sid, lambda r: pltpu.sync_copy(zrows_v, acc_sh.at[pl.ds(r, CH)]))
    plsc.subcore_barrier()

    @pl.loop(0, NSEG)
    def _(s):
        pltpu.sync_copy(col_hbm.at[wid].at[s], idx_v)

        @pl.loop(0, SEGW)
        def _(j):
            pltpu.sync_copy(ones_v, acc_sh.at[idx_v.at[j]], add=True)

    plsc.subcore_barrier()
    _chunk_loop(sid, lambda r: pltpu.sync_copy(
        acc_sh.at[pl.ds(r, CH)], out_hbm.at[cid].at[pl.ds(r, CH)]))


@functools.partial(
    pl.kernel,
    out_type=jax.ShapeDtypeStruct((NC, N, D), jnp.float32),
    mesh=_mesh,
    scratch_types=[
        pltpu.VMEM((SEGW, WIN), jnp.int32),
        pltpu.VMEM((SEGW, WIN), jnp.int32),
        pltpu.VMEM((2, WIN, D), jnp.float32),
        pltpu.VMEM_SHARED((N, D), jnp.float32),
        pltpu.SemaphoreType.DMA,
        pltpu.SemaphoreType.DMA,
    ],
)
def _agg_kernel(y_hbm, row_hbm, col_hbm, out_hbm, idxr_v, idxc_v, bufs_v,
                acc_sh, sema, semb):
    cid = lax.axis_index("c")
    sid = lax.axis_index("s")
    wid = cid * NS + sid
    bufa_v = bufs_v.at[0]
    bufb_v = bufs_v.at[1]
    v0 = jnp.zeros((16,), jnp.float32)

    @pl.loop(0, CH)
    def _(i):
        @pl.loop(0, D, step=16)
        def _(j):
            bufs_v[0, i, pl.ds(j, 16)] = v0

    _chunk_loop(sid, lambda r: pltpu.sync_copy(bufa_v, acc_sh.at[pl.ds(r, CH)]))
    plsc.subcore_barrier()

    def _start(j, buf, sem):
        pltpu.async_copy(y_hbm.at[idxr_v.at[j]], buf, sem)

    def _finish(j, buf, sem):
        pltpu.make_async_copy(y_hbm.at[idxr_v.at[j]], buf, sem).wait()
        pltpu.sync_copy(buf, acc_sh.at[idxc_v.at[j]], add=True)

    @pl.loop(0, NSEG)
    def _(s):
        pltpu.sync_copy(row_hbm.at[wid].at[s], idxr_v)
        pltpu.sync_copy(col_hbm.at[wid].at[s], idxc_v)
        _start(0, bufa_v, sema)

        @pl.loop(0, (SEGW + 1) // 2)
        def _(g):
            j = 2 * g

            @pl.when(j + 1 < SEGW)
            def _():
                _start(j + 1, bufb_v, semb)

            _finish(j, bufa_v, sema)

            @pl.when(j + 2 < SEGW)
            def _():
                _start(j + 2, bufa_v, sema)

            @pl.when(j + 1 < SEGW)
            def _():
                _finish(j + 1, bufb_v, semb)

    plsc.subcore_barrier()
    _chunk_loop(sid, lambda r: pltpu.sync_copy(
        acc_sh.at[pl.ds(r, CH)], out_hbm.at[cid].at[pl.ds(r, CH)]))


def _dot(a, b):
    return lax.dot_general(a, b, (((1,), (0,)), ((), ())),
                           precision=lax.Precision.HIGHEST,
                           preferred_element_type=jnp.float32)


def _dis_from_hist(hist_ref):
    deg = hist_ref[0, :, 0:1] + hist_ref[1, :, 0:1] + 1.0
    return lax.rsqrt(deg)


def _mm_body(x_ref, w_ref, o_ref):
    o_ref[...] = _dot(x_ref[...], w_ref[...])


def _scale_body(hist_ref, xw_ref, o_ref):
    o_ref[...] = xw_ref[...] * _dis_from_hist(hist_ref)


def _mid_body(hist_ref, s_ref, y_ref, w_ref, b_ref, o_ref):
    dis = _dis_from_hist(hist_ref)
    h = jnp.maximum(dis * (s_ref[0] + s_ref[1] + y_ref[...]) + b_ref[...], 0.0)
    o_ref[...] = dis * _dot(h, w_ref[...])


def _final_body(hist_ref, s_ref, y_ref, b_ref, o_ref):
    dis = _dis_from_hist(hist_ref)
    o_ref[...] = dis * (s_ref[0] + s_ref[1] + y_ref[...]) + b_ref[...]


_nd_f32 = jax.ShapeDtypeStruct((N, D), jnp.float32)


def kernel(x, edge_index, W1, b1, W2, b2):
    rc = edge_index.astype(jnp.int32).reshape(2, NW, NSEG, SEGW, WIN)
    row = rc[0]
    col = rc[1]
    b1r = b1.reshape(1, D)
    b2r = b2.reshape(1, D)

    hist = _hist_kernel(col)
    xw1 = pl.pallas_call(_mm_body, out_shape=_nd_f32)(x, W1)
    y1 = pl.pallas_call(_scale_body, out_shape=_nd_f32)(hist, xw1)
    s1 = _agg_kernel(y1, row, col)
    y2 = pl.pallas_call(_mid_body, out_shape=_nd_f32)(hist, s1, y1, W2, b1r)
    s2 = _agg_kernel(y2, row, col)
    out = pl.pallas_call(_final_body, out_shape=_nd_f32)(hist, s2, y2, b2r)
    return out


# speedup vs baseline: 3.0875x; 1.1188x over previous
import functools

import jax
import jax.numpy as jnp
from jax import lax
from jax.experimental import pallas as pl
from jax.experimental.pallas import tpu as pltpu
from jax.experimental.pallas import tpu_sc as plsc

N = 10000
D = 128
E = 320000
NC = 2
NS = 16
NW = NC * NS
EPW = E // NW
WIN = 80
SEGW = 25
NSEG = 5
CH = 80
NCH = N // CH
CPS = -(-NCH // NS)

_mesh = plsc.VectorSubcoreMesh(core_axis_name="c", subcore_axis_name="s")


def _fill(ref, rows, cols, val):
    v = jnp.full((16,), val, jnp.float32)

    @pl.loop(0, rows)
    def _(i):
        @pl.loop(0, cols, step=16)
        def _(j):
            ref[i, pl.ds(j, 16)] = v


def _chunk_loop(sid, body):
    @pl.loop(0, CPS)
    def _(k):
        c = k * NS + sid

        @pl.when(c < NCH)
        def _():
            body(c * CH)


@functools.partial(
    pl.kernel,
    out_type=jax.ShapeDtypeStruct((NC, N, D), jnp.float32),
    mesh=_mesh,
    scratch_types=[
        pltpu.VMEM((SEGW, WIN), jnp.int32),
        pltpu.VMEM((WIN, D), jnp.float32),
        pltpu.VMEM((CH, D), jnp.float32),
        pltpu.VMEM_SHARED((N, D), jnp.float32),
    ],
)
def _hist_kernel(col_hbm, out_hbm, idx_v, ones_v, zrows_v, acc_sh):
    cid = lax.axis_index("c")
    sid = lax.axis_index("s")
    wid = cid * NS + sid
    _fill(zrows_v, CH, D, 0.0)
    _fill(ones_v, WIN, D, 1.0)
    _chunk_loop(sid, lambda r: pltpu.sync_copy(zrows_v, acc_sh.at[pl.ds(r, CH)]))
    plsc.subcore_barrier()

    @pl.loop(0, NSEG)
    def _(s):
        pltpu.sync_copy(col_hbm.at[wid].at[s], idx_v)

        @pl.loop(0, SEGW)
        def _(j):
            pltpu.sync_copy(ones_v, acc_sh.at[idx_v.at[j]], add=True)

    plsc.subcore_barrier()
    _chunk_loop(sid, lambda r: pltpu.sync_copy(
        acc_sh.at[pl.ds(r, CH)], out_hbm.at[cid].at[pl.ds(r, CH)]))


@functools.partial(
    pl.kernel,
    out_type=jax.ShapeDtypeStruct((NC, N, D), jnp.float32),
    mesh=_mesh,
    scratch_types=[
        pltpu.VMEM((SEGW, WIN), jnp.int32),
        pltpu.VMEM((SEGW, WIN), jnp.int32),
        pltpu.VMEM((3, WIN, D), jnp.float32),
        pltpu.VMEM_SHARED((N, D), jnp.float32),
        pltpu.SemaphoreType.DMA,
        pltpu.SemaphoreType.DMA,
        pltpu.SemaphoreType.DMA,
    ],
)
def _agg_kernel(y_hbm, row_hbm, col_hbm, out_hbm, idxr_v, idxc_v, bufs_v,
                acc_sh, sema, semb, semc):
    cid = lax.axis_index("c")
    sid = lax.axis_index("s")
    wid = cid * NS + sid
    bufa_v = bufs_v.at[0]
    bufb_v = bufs_v.at[1]
    bufc_v = bufs_v.at[2]
    v0 = jnp.zeros((16,), jnp.float32)

    @pl.loop(0, CH)
    def _(i):
        @pl.loop(0, D, step=16)
        def _(j):
            bufs_v[0, i, pl.ds(j, 16)] = v0

    _chunk_loop(sid, lambda r: pltpu.sync_copy(bufa_v, acc_sh.at[pl.ds(r, CH)]))
    plsc.subcore_barrier()

    def _start(j, buf, sem):
        pltpu.async_copy(y_hbm.at[idxr_v.at[j]], buf, sem)

    def _finish(j, buf, sem):
        pltpu.make_async_copy(y_hbm.at[idxr_v.at[j]], buf, sem).wait()
        pltpu.sync_copy(buf, acc_sh.at[idxc_v.at[j]], add=True)

    @pl.loop(0, NSEG)
    def _(s):
        pltpu.sync_copy(row_hbm.at[wid].at[s], idxr_v)
        pltpu.sync_copy(col_hbm.at[wid].at[s], idxc_v)
        _start(0, bufa_v, sema)
        _start(1, bufb_v, semb)

        @pl.loop(0, (SEGW + 2) // 3)
        def _(g):
            j = 3 * g

            @pl.when(j + 2 < SEGW)
            def _():
                _start(j + 2, bufc_v, semc)

            _finish(j, bufa_v, sema)

            @pl.when(j + 3 < SEGW)
            def _():
                _start(j + 3, bufa_v, sema)

            @pl.when(j + 1 < SEGW)
            def _():
                _finish(j + 1, bufb_v, semb)

            @pl.when(j + 4 < SEGW)
            def _():
                _start(j + 4, bufb_v, semb)

            @pl.when(j + 2 < SEGW)
            def _():
                _finish(j + 2, bufc_v, semc)

    plsc.subcore_barrier()
    _chunk_loop(sid, lambda r: pltpu.sync_copy(
        acc_sh.at[pl.ds(r, CH)], out_hbm.at[cid].at[pl.ds(r, CH)]))


def _dot(a, b):
    return lax.dot_general(a, b, (((1,), (0,)), ((), ())),
                           precision=lax.Precision.HIGHEST,
                           preferred_element_type=jnp.float32)


def _dis_from_hist(hist_ref):
    deg = hist_ref[0, :, 0:1] + hist_ref[1, :, 0:1] + 1.0
    return lax.rsqrt(deg)


def _mm_body(x_ref, w_ref, o_ref):
    o_ref[...] = _dot(x_ref[...], w_ref[...])


def _scale_body(hist_ref, xw_ref, o_ref):
    o_ref[...] = xw_ref[...] * _dis_from_hist(hist_ref)


def _mid_body(hist_ref, s_ref, y_ref, w_ref, b_ref, o_ref):
    dis = _dis_from_hist(hist_ref)
    h = jnp.maximum(dis * (s_ref[0] + s_ref[1] + y_ref[...]) + b_ref[...], 0.0)
    o_ref[...] = dis * _dot(h, w_ref[...])


def _final_body(hist_ref, s_ref, y_ref, b_ref, o_ref):
    dis = _dis_from_hist(hist_ref)
    o_ref[...] = dis * (s_ref[0] + s_ref[1] + y_ref[...]) + b_ref[...]


_nd_f32 = jax.ShapeDtypeStruct((N, D), jnp.float32)


def kernel(x, edge_index, W1, b1, W2, b2):
    rc = edge_index.astype(jnp.int32).reshape(2, NW, NSEG, SEGW, WIN)
    row = rc[0]
    col = rc[1]
    b1r = b1.reshape(1, D)
    b2r = b2.reshape(1, D)

    hist = _hist_kernel(col)
    xw1 = pl.pallas_call(_mm_body, out_shape=_nd_f32)(x, W1)
    y1 = pl.pallas_call(_scale_body, out_shape=_nd_f32)(hist, xw1)
    s1 = _agg_kernel(y1, row, col)
    y2 = pl.pallas_call(_mid_body, out_shape=_nd_f32)(hist, s1, y1, W2, b1r)
    s2 = _agg_kernel(y2, row, col)
    out = pl.pallas_call(_final_body, out_shape=_nd_f32)(hist, s2, y2, b2r)
    return out


# speedup vs baseline: 3.0894x; 1.0006x over previous
import functools

import jax
import jax.numpy as jnp
from jax import lax
from jax.experimental import pallas as pl
from jax.experimental.pallas import tpu as pltpu
from jax.experimental.pallas import tpu_sc as plsc

N = 10000
D = 128
E = 320000
NC = 2
NS = 16
NW = NC * NS
EPW = E // NW
WIN = 80
SEGW = 25
NSEG = 5
CH = 80
NCH = N // CH
CPS = -(-NCH // NS)

_mesh = plsc.VectorSubcoreMesh(core_axis_name="c", subcore_axis_name="s")


def _fill(ref, rows, cols, val):
    v = jnp.full((16,), val, jnp.float32)

    @pl.loop(0, rows)
    def _(i):
        @pl.loop(0, cols, step=16)
        def _(j):
            ref[i, pl.ds(j, 16)] = v


def _chunk_loop(sid, body):
    @pl.loop(0, CPS)
    def _(k):
        c = k * NS + sid

        @pl.when(c < NCH)
        def _():
            body(c * CH)


@functools.partial(
    pl.kernel,
    out_type=jax.ShapeDtypeStruct((NC, N, D), jnp.float32),
    mesh=_mesh,
    scratch_types=[
        pltpu.VMEM((SEGW, WIN), jnp.int32),
        pltpu.VMEM((WIN, D), jnp.float32),
        pltpu.VMEM((CH, D), jnp.float32),
        pltpu.VMEM_SHARED((N, D), jnp.float32),
    ],
)
def _hist_kernel(col_hbm, out_hbm, idx_v, ones_v, zrows_v, acc_sh):
    cid = lax.axis_index("c")
    sid = lax.axis_index("s")
    wid = cid * NS + sid
    _fill(zrows_v, CH, D, 0.0)
    _fill(ones_v, WIN, D, 1.0)
    _chunk_loop(sid, lambda r: pltpu.sync_copy(zrows_v, acc_sh.at[pl.ds(r, CH)]))
    plsc.subcore_barrier()

    @pl.loop(0, NSEG)
    def _(s):
        pltpu.sync_copy(col_hbm.at[wid].at[s], idx_v)

        @pl.loop(0, SEGW)
        def _(j):
            pltpu.sync_copy(ones_v, acc_sh.at[idx_v.at[j]], add=True)

    plsc.subcore_barrier()
    _chunk_loop(sid, lambda r: pltpu.sync_copy(
        acc_sh.at[pl.ds(r, CH)], out_hbm.at[cid].at[pl.ds(r, CH)]))


@functools.partial(
    pl.kernel,
    out_type=jax.ShapeDtypeStruct((NC, N, D), jnp.float32),
    mesh=_mesh,
    scratch_types=[
        pltpu.VMEM((SEGW, WIN), jnp.int32),
        pltpu.VMEM((SEGW, WIN), jnp.int32),
        pltpu.VMEM((4, WIN, D), jnp.float32),
        pltpu.VMEM_SHARED((N, D), jnp.float32),
        pltpu.SemaphoreType.DMA,
        pltpu.SemaphoreType.DMA,
        pltpu.SemaphoreType.DMA,
        pltpu.SemaphoreType.DMA,
    ],
)
def _agg_kernel(y_hbm, row_hbm, col_hbm, out_hbm, idxr_v, idxc_v, bufs_v,
                acc_sh, sema, semb, semc, semd):
    cid = lax.axis_index("c")
    sid = lax.axis_index("s")
    wid = cid * NS + sid
    bufa_v = bufs_v.at[0]
    bufb_v = bufs_v.at[1]
    bufc_v = bufs_v.at[2]
    bufd_v = bufs_v.at[3]
    v0 = jnp.zeros((16,), jnp.float32)

    @pl.loop(0, CH)
    def _(i):
        @pl.loop(0, D, step=16)
        def _(j):
            bufs_v[0, i, pl.ds(j, 16)] = v0

    _chunk_loop(sid, lambda r: pltpu.sync_copy(bufa_v, acc_sh.at[pl.ds(r, CH)]))
    plsc.subcore_barrier()

    def _start(j, buf, sem):
        pltpu.async_copy(y_hbm.at[idxr_v.at[j]], buf, sem)

    def _finish(j, buf, sem):
        pltpu.make_async_copy(y_hbm.at[idxr_v.at[j]], buf, sem).wait()
        pltpu.sync_copy(buf, acc_sh.at[idxc_v.at[j]], add=True)

    @pl.loop(0, NSEG)
    def _(s):
        pltpu.sync_copy(row_hbm.at[wid].at[s], idxr_v)
        pltpu.sync_copy(col_hbm.at[wid].at[s], idxc_v)
        _start(0, bufa_v, sema)
        _start(1, bufb_v, semb)
        _start(2, bufc_v, semc)

        @pl.loop(0, (SEGW + 3) // 4)
        def _(g):
            j = 4 * g

            @pl.when(j + 3 < SEGW)
            def _():
                _start(j + 3, bufd_v, semd)

            _finish(j, bufa_v, sema)

            @pl.when(j + 4 < SEGW)
            def _():
                _start(j + 4, bufa_v, sema)

            @pl.when(j + 1 < SEGW)
            def _():
                _finish(j + 1, bufb_v, semb)

            @pl.when(j + 5 < SEGW)
            def _():
                _start(j + 5, bufb_v, semb)

            @pl.when(j + 2 < SEGW)
            def _():
                _finish(j + 2, bufc_v, semc)

            @pl.when(j + 6 < SEGW)
            def _():
                _start(j + 6, bufc_v, semc)

            @pl.when(j + 3 < SEGW)
            def _():
                _finish(j + 3, bufd_v, semd)

    plsc.subcore_barrier()
    _chunk_loop(sid, lambda r: pltpu.sync_copy(
        acc_sh.at[pl.ds(r, CH)], out_hbm.at[cid].at[pl.ds(r, CH)]))


def _dot(a, b):
    return lax.dot_general(a, b, (((1,), (0,)), ((), ())),
                           precision=lax.Precision.HIGHEST,
                           preferred_element_type=jnp.float32)


def _dis_from_hist(hist_ref):
    deg = hist_ref[0, :, 0:1] + hist_ref[1, :, 0:1] + 1.0
    return lax.rsqrt(deg)


def _mm_body(x_ref, w_ref, o_ref):
    o_ref[...] = _dot(x_ref[...], w_ref[...])


def _scale_body(hist_ref, xw_ref, o_ref):
    o_ref[...] = xw_ref[...] * _dis_from_hist(hist_ref)


def _mid_body(hist_ref, s_ref, y_ref, w_ref, b_ref, o_ref):
    dis = _dis_from_hist(hist_ref)
    h = jnp.maximum(dis * (s_ref[0] + s_ref[1] + y_ref[...]) + b_ref[...], 0.0)
    o_ref[...] = dis * _dot(h, w_ref[...])


def _final_body(hist_ref, s_ref, y_ref, b_ref, o_ref):
    dis = _dis_from_hist(hist_ref)
    o_ref[...] = dis * (s_ref[0] + s_ref[1] + y_ref[...]) + b_ref[...]


_nd_f32 = jax.ShapeDtypeStruct((N, D), jnp.float32)


def kernel(x, edge_index, W1, b1, W2, b2):
    rc = edge_index.astype(jnp.int32).reshape(2, NW, NSEG, SEGW, WIN)
    row = rc[0]
    col = rc[1]
    b1r = b1.reshape(1, D)
    b2r = b2.reshape(1, D)

    hist = _hist_kernel(col)
    xw1 = pl.pallas_call(_mm_body, out_shape=_nd_f32)(x, W1)
    y1 = pl.pallas_call(_scale_body, out_shape=_nd_f32)(hist, xw1)
    s1 = _agg_kernel(y1, row, col)
    y2 = pl.pallas_call(_mid_body, out_shape=_nd_f32)(hist, s1, y1, W2, b1r)
    s2 = _agg_kernel(y2, row, col)
    out = pl.pallas_call(_final_body, out_shape=_nd_f32)(hist, s2, y2, b2r)
    return out
